# baseline (speedup 1.0000x reference)
import functools

import jax
import jax.numpy as jnp
from jax.experimental import pallas as pl
from jax.experimental.pallas import tpu as pltpu

_TOPK = 8


def _gate_body(xa_ref, xb_ref, wta_ref, wtb_ref, bias_ref, w_out_ref,
               i_out_ref, *, n_experts):
    scores = (jnp.dot(xa_ref[...], wta_ref[...],
                      preferred_element_type=jnp.float32)
              + jnp.dot(xb_ref[...], wtb_ref[...],
                        preferred_element_type=jnp.float32))
    scores = scores + bias_ref[...]

    st = scores.T
    idx = jax.lax.broadcasted_iota(jnp.int32, st.shape, 0).astype(jnp.float32)
    vals = st
    top_vs = []
    top_is = []
    for _ in range(_TOPK):
        m = jnp.max(vals, axis=0, keepdims=True)
        am = jnp.min(jnp.where(vals == m, idx, float(n_experts)), axis=0,
                     keepdims=True)
        top_vs.append(m)
        top_is.append(am)
        vals = jnp.where(idx == am, -jnp.inf, vals)

    tv = jnp.concatenate(top_vs, axis=0)
    ti = jnp.concatenate(top_is, axis=0)
    e = jnp.exp(tv - tv[0:1])
    w = e / jnp.sum(e, axis=0, keepdims=True)
    w_out_ref[...] = w
    i_out_ref[...] = ti.astype(jnp.int32)


@functools.partial(jax.jit, static_argnames=("block_rows",))
def _moe_gate(x, weight, bias, block_rows=1024):
    n_rows, k = x.shape
    n_experts = weight.shape[0]
    wt = weight.T
    bias2d = bias.reshape(1, n_experts)

    grid = (n_rows // block_rows,)
    out_w, out_i = pl.pallas_call(
        functools.partial(_gate_body, n_experts=n_experts),
        grid=grid,
        in_specs=[
            pl.BlockSpec((block_rows, k // 2), lambda i: (i, 0)),
            pl.BlockSpec((block_rows, k // 2), lambda i: (i, 1)),
            pl.BlockSpec((k // 2, n_experts), lambda i: (0, 0)),
            pl.BlockSpec((k // 2, n_experts), lambda i: (1, 0)),
            pl.BlockSpec((1, n_experts), lambda i: (0, 0)),
        ],
        out_specs=[
            pl.BlockSpec((_TOPK, block_rows), lambda i: (0, i)),
            pl.BlockSpec((_TOPK, block_rows), lambda i: (0, i)),
        ],
        out_shape=[
            jax.ShapeDtypeStruct((_TOPK, n_rows), jnp.float32),
            jax.ShapeDtypeStruct((_TOPK, n_rows), jnp.int32),
        ],
        compiler_params=pltpu.CompilerParams(
            dimension_semantics=("parallel",),
        ),
    )(x, x, wt, wt, bias2d)
    return out_w.T, out_i.T


def kernel(x, weight, bias):
    w, i = _moe_gate(x, weight, bias)
    return w.astype(x.dtype), i


# speedup vs baseline: 1.0126x; 1.0126x over previous
import functools

import jax
import jax.numpy as jnp
from jax.experimental import pallas as pl
from jax.experimental.pallas import tpu as pltpu

_TOPK = 8


def _gate_body(x_ref, wt_ref, bias_ref, w_out_ref, i_out_ref, *, n_experts):
    x = x_ref[...]
    wt = wt_ref[...]
    scores = jnp.dot(x, wt, preferred_element_type=jnp.float32)
    scores = scores + bias_ref[...]

    st = scores.T
    idx = jax.lax.broadcasted_iota(jnp.int32, st.shape, 0).astype(jnp.float32)
    vals = st
    top_vs = []
    top_is = []
    for _ in range(_TOPK):
        m = jnp.max(vals, axis=0, keepdims=True)
        am = jnp.min(jnp.where(vals == m, idx, float(n_experts)), axis=0,
                     keepdims=True)
        top_vs.append(m)
        top_is.append(am)
        vals = jnp.where(idx == am, -jnp.inf, vals)

    tv = jnp.concatenate(top_vs, axis=0)
    ti = jnp.concatenate(top_is, axis=0)
    e = jnp.exp(tv - tv[0:1])
    w = e / jnp.sum(e, axis=0, keepdims=True)
    w_out_ref[...] = w
    i_out_ref[...] = ti.astype(jnp.int32)


@functools.partial(jax.jit, static_argnames=("block_rows",))
def _moe_gate(x, weight, bias, block_rows=1024):
    n_rows, k = x.shape
    n_experts = weight.shape[0]
    wt = weight.T
    bias2d = bias.reshape(1, n_experts)

    grid = (n_rows // block_rows,)
    out_w, out_i = pl.pallas_call(
        functools.partial(_gate_body, n_experts=n_experts),
        grid=grid,
        in_specs=[
            pl.BlockSpec((block_rows, k), lambda i: (i, 0)),
            pl.BlockSpec((k, n_experts), lambda i: (0, 0)),
            pl.BlockSpec((1, n_experts), lambda i: (0, 0)),
        ],
        out_specs=[
            pl.BlockSpec((_TOPK, block_rows), lambda i: (0, i)),
            pl.BlockSpec((_TOPK, block_rows), lambda i: (0, i)),
        ],
        out_shape=[
            jax.ShapeDtypeStruct((_TOPK, n_rows), jnp.float32),
            jax.ShapeDtypeStruct((_TOPK, n_rows), jnp.int32),
        ],
        compiler_params=pltpu.CompilerParams(
            dimension_semantics=("arbitrary",),
        ),
    )(x, wt, bias2d)
    return out_w.T, out_i.T


def kernel(x, weight, bias):
    w, i = _moe_gate(x, weight, bias)
    return w.astype(x.dtype), i
